# baseline (speedup 1.0000x reference)
import jax
import jax.numpy as jnp
import numpy as _np
from jax import lax
from jax.experimental import pallas as pl
from jax.experimental.pallas import tpu as pltpu
from jax.experimental.pallas import tpu_sc as plsc

N, E, D, H, O = 10000, 320000, 128, 128, 64
HW = H // 2
NC, NS = 2, 16
NW = NC * NS
B = 128
EPAD = 323584
EP = EPAD // NW
NB = EP // B
NPAD = 10240
RPT = NPAD // NS
WDEG = 16
NBLK = 2000
DEPTH = 6
F32 = jnp.float32


def _make_sc_agg(two_sweeps, with_deg):
    mesh = plsc.VectorSubcoreMesh(core_axis_name="c", subcore_axis_name="s")
    out_type = [jax.ShapeDtypeStruct((NC, NPAD, H), F32)]
    scratch = [
        pltpu.VMEM((NB, B), jnp.int32),
        pltpu.VMEM((NB, B), jnp.int32),
    ] + [pltpu.VMEM((B, HW), F32)] * DEPTH + [
        pltpu.VMEM_SHARED((NPAD, HW), F32),
    ] + [pltpu.SemaphoreType.DMA] * DEPTH + [
    ]
    if with_deg:
        out_type.append(jax.ShapeDtypeStruct((NC, NPAD, WDEG), F32))
        scratch += [
            pltpu.VMEM((B, WDEG), F32),
            pltpu.VMEM_SHARED((NPAD, WDEG), F32),
        ]

    def body(*refs):
        y, src4, dst3, zw = refs[:4]
        rest = refs[4:]
        if with_deg:
            (zdeg, ones_h, p_out, d_out, sidx, didx) = rest[:6]
            rbufs = rest[6:6 + DEPTH]
            acc = rest[6 + DEPTH]
            sems = rest[7 + DEPTH:7 + 2 * DEPTH]
            ones_v, dacc = rest[7 + 2 * DEPTH:]
        else:
            p_out, sidx, didx = rest[:3]
            rbufs = rest[3:3 + DEPTH]
            acc = rest[3 + DEPTH]
            sems = rest[4 + DEPTH:4 + 2 * DEPTH]
        bufs = tuple(zip(rbufs, sems))
        c = lax.axis_index("c")
        s = lax.axis_index("s")
        wid = s * NC + c
        rows = pl.ds(s * RPT, RPT)

        pltpu.sync_copy(zw, acc.at[rows])
        if with_deg:
            pltpu.sync_copy(zdeg, dacc.at[rows])
            pltpu.sync_copy(ones_h, ones_v)
        pltpu.sync_copy(src4.at[0, wid], sidx)
        pltpu.sync_copy(dst3.at[wid], didx)
        plsc.subcore_barrier()

        def sweep(count_deg):
            def start(i, buf, sem):
                pltpu.async_copy(y.at[sidx.at[i]], buf, sem)

            def finish(i, buf, sem):
                pltpu.make_async_copy(y.at[sidx.at[i]], buf, sem).wait()
                pltpu.sync_copy(buf, acc.at[didx.at[i]], add=True)
                if count_deg:
                    pltpu.sync_copy(ones_v, dacc.at[didx.at[i]], add=True)

            depth = len(bufs)
            nmain = (NB - (depth - 1)) // depth
            for j in range(depth - 1):
                start(j, *bufs[j])

            def ring(k, carry):
                base = depth * k
                for j in range(depth):
                    i = base + j
                    start(i + depth - 1, *bufs[(j + depth - 1) % depth])
                    finish(i, *bufs[j])
                return carry

            lax.fori_loop(0, nmain, ring, 0)
            for i in range(nmain * depth, NB):
                nxt = i + depth - 1
                if nxt < NB:
                    start(nxt, *bufs[nxt % depth])
                finish(i, *bufs[i % depth])

        sweep(with_deg)
        plsc.subcore_barrier()
        pltpu.sync_copy(acc.at[rows], p_out.at[c, rows, pl.ds(0, HW)])
        if with_deg:
            pltpu.sync_copy(dacc.at[rows], d_out.at[c, rows])
        if two_sweeps:
            pltpu.sync_copy(zw, acc.at[rows])
            pltpu.sync_copy(src4.at[1, wid], sidx)
            plsc.subcore_barrier()
            sweep(False)
            plsc.subcore_barrier()
            pltpu.sync_copy(acc.at[rows], p_out.at[c, rows, pl.ds(HW, HW)])

    return pl.kernel(body, out_type=out_type, mesh=mesh, scratch_types=scratch,
                     compiler_params=pltpu.CompilerParams(
                         use_tc_tiling_on_sc=False))


_sc_agg_deg = _make_sc_agg(True, True)
_sc_agg_h = _make_sc_agg(True, False)
_sc_agg_o = _make_sc_agg(False, False)


def _row_spec(w):
    return pl.BlockSpec((NBLK, w), lambda i: (i, 0))


def _full_spec(shape):
    nd = len(shape)
    return pl.BlockSpec(shape, lambda i, _nd=nd: (0,) * _nd)


def _part_spec(w):
    return pl.BlockSpec((NC, NBLK, w), lambda i: (0, i, 0))


def _inv_deg(dg_ref):
    deg = dg_ref[0, :, 0:1] + dg_ref[1, :, 0:1]
    return 1.0 / jnp.maximum(deg, 1.0)


def _mm(a, w):
    return lax.dot_general(a, w, (((1,), (1,)), ((), ())),
                           preferred_element_type=F32)


def _tc1(x_r, wl_r, wr_r, bl_r, y_r, r_r):
    xb = x_r[...]
    y_r[...] = _mm(xb, wl_r[...])
    r_r[...] = _mm(xb, wr_r[...]) + bl_r[...]


_tc1_call = pl.pallas_call(
    _tc1,
    grid=(N // NBLK,),
    in_specs=[_row_spec(D), _full_spec((H, D)), _full_spec((H, D)),
              _full_spec((1, H))],
    out_specs=[_row_spec(H), _row_spec(H)],
    out_shape=[jax.ShapeDtypeStruct((N, H), F32)] * 2,
)


def _tc2(p_r, dg_r, r0_r, s_r, b_r, wl_r, wr_r, bl_r, y_r, r_r):
    sage = (p_r[0] + p_r[1]) * _inv_deg(dg_r) + r0_r[...]
    h = jnp.maximum(sage * s_r[...] + b_r[...], 0.0)
    y_r[...] = _mm(h, wl_r[...])
    r_r[...] = _mm(h, wr_r[...]) + bl_r[...]


_tc2_call = pl.pallas_call(
    _tc2,
    grid=(N // NBLK,),
    in_specs=[_part_spec(H), _part_spec(WDEG), _row_spec(H),
              _full_spec((1, H)), _full_spec((1, H)),
              _full_spec((H, H)), _full_spec((H, H)), _full_spec((1, H))],
    out_specs=[_row_spec(H), _row_spec(H)],
    out_shape=[jax.ShapeDtypeStruct((N, H), F32)] * 2,
)


def _tc3(q_r, dg_r, r1_r, s_r, b_r, af_r, wh_r, bh_r,
         wl2_r, wr2_r, bl2_r, yr_r):
    sage = (q_r[0] + q_r[1]) * _inv_deg(dg_r) + r1_r[...]
    h1 = jnp.maximum(sage * s_r[...] + b_r[...], 0.0)
    af = _mm(af_r[...], wh_r[...]) + bh_r[...]
    cat = jnp.concatenate([h1, af], axis=-1)
    yr_r[...] = jnp.concatenate(
        [_mm(cat, wl2_r[...]), _mm(cat, wr2_r[...]) + bl2_r[...]], axis=-1)


_tc3_call = pl.pallas_call(
    _tc3,
    grid=(N // NBLK,),
    in_specs=[_part_spec(H), _part_spec(WDEG), _row_spec(H),
              _full_spec((1, H)), _full_spec((1, H)),
              _row_spec(D), _full_spec((H, D)), _full_spec((1, H)),
              _full_spec((O, 2 * H)), _full_spec((O, 2 * H)),
              _full_spec((1, O))],
    out_specs=_row_spec(H),
    out_shape=jax.ShapeDtypeStruct((N, H), F32),
)


def _tc4(p2_r, dg_r, yr_r, o_r):
    z = ((p2_r[0, :, :O] + p2_r[1, :, :O]) * _inv_deg(dg_r)
         + yr_r[:, O:])
    m = jnp.max(z, axis=-1, keepdims=True)
    ez = jnp.exp(z - m)
    o_r[...] = z - m - jnp.log(jnp.sum(ez, axis=-1, keepdims=True))


_tc4_call = pl.pallas_call(
    _tc4,
    grid=(N // NBLK,),
    in_specs=[_part_spec(H), _part_spec(WDEG), _row_spec(H)],
    out_specs=_row_spec(O),
    out_shape=jax.ShapeDtypeStruct((N, O), F32),
)


def kernel(x, edge_index, Wl0, bl0, Wr0, Wl1, bl1, Wr1, Wl2, bl2, Wr2,
           g0, b0, g1, b1, Wh, bh, add_feat):
    if EPAD != E:
        pad_src = jnp.asarray(_np.arange(EPAD - E, dtype=_np.int32) % N)
        pad_dst = jnp.asarray(
            N + _np.arange(EPAD - E, dtype=_np.int32) % (NPAD - N))
        src = jnp.concatenate([edge_index[0], pad_src])
        dst = jnp.concatenate([edge_index[1], pad_dst])
    else:
        src, dst = edge_index[0], edge_index[1]
    s2 = src + src
    src4 = jnp.stack([s2, s2 + 1]).reshape(2, NW, NB, B)
    dst3 = dst.reshape(NW, NB, B)
    z_hw = jnp.zeros((RPT, HW), F32)
    z_deg = jnp.zeros((RPT, WDEG), F32)
    ones_h = jnp.ones((B, WDEG), F32)
    bn_s0 = (g0 / jnp.sqrt(1.0 + 1e-5)).reshape(1, H)
    bn_s1 = (g1 / jnp.sqrt(1.0 + 1e-5)).reshape(1, H)

    y0, r0 = _tc1_call(x, Wl0, Wr0, bl0.reshape(1, H))
    p0, dg = _sc_agg_deg(y0.reshape(2 * N, HW), src4, dst3, z_hw,
                         z_deg, ones_h)
    y1, r1 = _tc2_call(p0, dg, r0, bn_s0, b0.reshape(1, H),
                       Wl1, Wr1, bl1.reshape(1, H))
    (p1,) = _sc_agg_h(y1.reshape(2 * N, HW), src4, dst3, z_hw)
    yr2 = _tc3_call(p1, dg, r1, bn_s1, b1.reshape(1, H),
                    add_feat, Wh, bh.reshape(1, H),
                    Wl2, Wr2, bl2.reshape(1, O))
    (p2,) = _sc_agg_o(yr2.reshape(2 * N, HW), src4, dst3, z_hw)
    return _tc4_call(p2, dg, yr2)


# speedup vs baseline: 1.0019x; 1.0019x over previous
import jax
import jax.numpy as jnp
import numpy as _np
from jax import lax
from jax.experimental import pallas as pl
from jax.experimental.pallas import tpu as pltpu
from jax.experimental.pallas import tpu_sc as plsc

N, E, D, H, O = 10000, 320000, 128, 128, 64
HW = H // 2
NC, NS = 2, 16
NW = NC * NS
B = 128
EPAD = 323584
EP = EPAD // NW
NB = EP // B
NPAD = 10240
RPT = NPAD // NS
WDEG = 16
NBLK = 2000
DEPTH = 4
F32 = jnp.float32


def _make_sc_agg(two_sweeps, with_deg):
    mesh = plsc.VectorSubcoreMesh(core_axis_name="c", subcore_axis_name="s")
    out_type = [jax.ShapeDtypeStruct((NC, NPAD, H), F32)]
    scratch = [
        pltpu.VMEM((NB, B), jnp.int32),
        pltpu.VMEM((NB, B), jnp.int32),
    ] + [pltpu.VMEM((B, HW), F32)] * DEPTH + [
        pltpu.VMEM_SHARED((NPAD, HW), F32),
    ] + [pltpu.SemaphoreType.DMA] * DEPTH + [
    ]
    if with_deg:
        out_type.append(jax.ShapeDtypeStruct((NC, NPAD, WDEG), F32))
        scratch += [
            pltpu.VMEM((B, WDEG), F32),
            pltpu.VMEM_SHARED((NPAD, WDEG), F32),
        ]

    def body(*refs):
        y, src4, dst3, zw = refs[:4]
        rest = refs[4:]
        if with_deg:
            (zdeg, ones_h, p_out, d_out, sidx, didx) = rest[:6]
            rbufs = rest[6:6 + DEPTH]
            acc = rest[6 + DEPTH]
            sems = rest[7 + DEPTH:7 + 2 * DEPTH]
            ones_v, dacc = rest[7 + 2 * DEPTH:]
        else:
            p_out, sidx, didx = rest[:3]
            rbufs = rest[3:3 + DEPTH]
            acc = rest[3 + DEPTH]
            sems = rest[4 + DEPTH:4 + 2 * DEPTH]
        bufs = tuple(zip(rbufs, sems))
        c = lax.axis_index("c")
        s = lax.axis_index("s")
        wid = s * NC + c
        rows = pl.ds(s * RPT, RPT)

        pltpu.sync_copy(zw, acc.at[rows])
        if with_deg:
            pltpu.sync_copy(zdeg, dacc.at[rows])
            pltpu.sync_copy(ones_h, ones_v)
        pltpu.sync_copy(src4.at[0, wid], sidx)
        pltpu.sync_copy(dst3.at[wid], didx)
        plsc.subcore_barrier()

        def sweep(count_deg):
            def start(i, buf, sem):
                pltpu.async_copy(y.at[sidx.at[i]], buf, sem)

            def finish(i, buf, sem):
                pltpu.make_async_copy(y.at[sidx.at[i]], buf, sem).wait()
                pltpu.sync_copy(buf, acc.at[didx.at[i]], add=True)
                if count_deg:
                    pltpu.sync_copy(ones_v, dacc.at[didx.at[i]], add=True)

            depth = len(bufs)
            nmain = (NB - (depth - 1)) // depth
            for j in range(depth - 1):
                start(j, *bufs[j])

            def ring(k, carry):
                base = depth * k
                for j in range(depth):
                    i = base + j
                    start(i + depth - 1, *bufs[(j + depth - 1) % depth])
                    finish(i, *bufs[j])
                return carry

            lax.fori_loop(0, nmain, ring, 0)
            for i in range(nmain * depth, NB):
                nxt = i + depth - 1
                if nxt < NB:
                    start(nxt, *bufs[nxt % depth])
                finish(i, *bufs[i % depth])

        sweep(with_deg)
        plsc.subcore_barrier()
        pltpu.sync_copy(acc.at[rows], p_out.at[c, rows, pl.ds(0, HW)])
        if with_deg:
            pltpu.sync_copy(dacc.at[rows], d_out.at[c, rows])
        if two_sweeps:
            pltpu.sync_copy(zw, acc.at[rows])
            pltpu.sync_copy(src4.at[1, wid], sidx)
            plsc.subcore_barrier()
            sweep(False)
            plsc.subcore_barrier()
            pltpu.sync_copy(acc.at[rows], p_out.at[c, rows, pl.ds(HW, HW)])

    return pl.kernel(body, out_type=out_type, mesh=mesh, scratch_types=scratch,
                     compiler_params=pltpu.CompilerParams(
                         use_tc_tiling_on_sc=False))


_sc_agg_deg = _make_sc_agg(True, True)
_sc_agg_h = _make_sc_agg(True, False)
_sc_agg_o = _make_sc_agg(False, False)


def _row_spec(w):
    return pl.BlockSpec((NBLK, w), lambda i: (i, 0))


def _full_spec(shape):
    nd = len(shape)
    return pl.BlockSpec(shape, lambda i, _nd=nd: (0,) * _nd)


def _part_spec(w):
    return pl.BlockSpec((NC, NBLK, w), lambda i: (0, i, 0))


def _inv_deg(dg_ref):
    deg = dg_ref[0, :, 0:1] + dg_ref[1, :, 0:1]
    return 1.0 / jnp.maximum(deg, 1.0)


def _mm(a, w):
    return lax.dot_general(a, w, (((1,), (1,)), ((), ())),
                           preferred_element_type=F32)


def _tc1(x_r, wl_r, wr_r, bl_r, y_r, r_r):
    xb = x_r[...]
    y_r[...] = _mm(xb, wl_r[...])
    r_r[...] = _mm(xb, wr_r[...]) + bl_r[...]


_tc1_call = pl.pallas_call(
    _tc1,
    grid=(N // NBLK,),
    in_specs=[_row_spec(D), _full_spec((H, D)), _full_spec((H, D)),
              _full_spec((1, H))],
    out_specs=[_row_spec(H), _row_spec(H)],
    out_shape=[jax.ShapeDtypeStruct((N, H), F32)] * 2,
)


def _tc2(p_r, dg_r, r0_r, s_r, b_r, wl_r, wr_r, bl_r, y_r, r_r):
    sage = (p_r[0] + p_r[1]) * _inv_deg(dg_r) + r0_r[...]
    h = jnp.maximum(sage * s_r[...] + b_r[...], 0.0)
    y_r[...] = _mm(h, wl_r[...])
    r_r[...] = _mm(h, wr_r[...]) + bl_r[...]


_tc2_call = pl.pallas_call(
    _tc2,
    grid=(N // NBLK,),
    in_specs=[_part_spec(H), _part_spec(WDEG), _row_spec(H),
              _full_spec((1, H)), _full_spec((1, H)),
              _full_spec((H, H)), _full_spec((H, H)), _full_spec((1, H))],
    out_specs=[_row_spec(H), _row_spec(H)],
    out_shape=[jax.ShapeDtypeStruct((N, H), F32)] * 2,
)


def _tc3(q_r, dg_r, r1_r, s_r, b_r, af_r, wh_r, bh_r,
         wl2_r, wr2_r, bl2_r, yr_r):
    sage = (q_r[0] + q_r[1]) * _inv_deg(dg_r) + r1_r[...]
    h1 = jnp.maximum(sage * s_r[...] + b_r[...], 0.0)
    af = _mm(af_r[...], wh_r[...]) + bh_r[...]
    cat = jnp.concatenate([h1, af], axis=-1)
    yr_r[...] = jnp.concatenate(
        [_mm(cat, wl2_r[...]), _mm(cat, wr2_r[...]) + bl2_r[...]], axis=-1)


_tc3_call = pl.pallas_call(
    _tc3,
    grid=(N // NBLK,),
    in_specs=[_part_spec(H), _part_spec(WDEG), _row_spec(H),
              _full_spec((1, H)), _full_spec((1, H)),
              _row_spec(D), _full_spec((H, D)), _full_spec((1, H)),
              _full_spec((O, 2 * H)), _full_spec((O, 2 * H)),
              _full_spec((1, O))],
    out_specs=_row_spec(H),
    out_shape=jax.ShapeDtypeStruct((N, H), F32),
)


def _tc4(p2_r, dg_r, yr_r, o_r):
    z = ((p2_r[0, :, :O] + p2_r[1, :, :O]) * _inv_deg(dg_r)
         + yr_r[:, O:])
    m = jnp.max(z, axis=-1, keepdims=True)
    ez = jnp.exp(z - m)
    o_r[...] = z - m - jnp.log(jnp.sum(ez, axis=-1, keepdims=True))


_tc4_call = pl.pallas_call(
    _tc4,
    grid=(N // NBLK,),
    in_specs=[_part_spec(H), _part_spec(WDEG), _row_spec(H)],
    out_specs=_row_spec(O),
    out_shape=jax.ShapeDtypeStruct((N, O), F32),
)


def kernel(x, edge_index, Wl0, bl0, Wr0, Wl1, bl1, Wr1, Wl2, bl2, Wr2,
           g0, b0, g1, b1, Wh, bh, add_feat):
    if EPAD != E:
        pad_src = jnp.asarray(_np.arange(EPAD - E, dtype=_np.int32) % N)
        pad_dst = jnp.asarray(
            N + _np.arange(EPAD - E, dtype=_np.int32) % (NPAD - N))
        src = jnp.concatenate([edge_index[0], pad_src])
        dst = jnp.concatenate([edge_index[1], pad_dst])
    else:
        src, dst = edge_index[0], edge_index[1]
    s2 = src + src
    src4 = jnp.stack([s2, s2 + 1]).reshape(2, NW, NB, B)
    dst3 = dst.reshape(NW, NB, B)
    z_hw = jnp.zeros((RPT, HW), F32)
    z_deg = jnp.zeros((RPT, WDEG), F32)
    ones_h = jnp.ones((B, WDEG), F32)
    bn_s0 = (g0 / jnp.sqrt(1.0 + 1e-5)).reshape(1, H)
    bn_s1 = (g1 / jnp.sqrt(1.0 + 1e-5)).reshape(1, H)

    y0, r0 = _tc1_call(x, Wl0, Wr0, bl0.reshape(1, H))
    p0, dg = _sc_agg_deg(y0.reshape(2 * N, HW), src4, dst3, z_hw,
                         z_deg, ones_h)
    y1, r1 = _tc2_call(p0, dg, r0, bn_s0, b0.reshape(1, H),
                       Wl1, Wr1, bl1.reshape(1, H))
    (p1,) = _sc_agg_h(y1.reshape(2 * N, HW), src4, dst3, z_hw)
    yr2 = _tc3_call(p1, dg, r1, bn_s1, b1.reshape(1, H),
                    add_feat, Wh, bh.reshape(1, H),
                    Wl2, Wr2, bl2.reshape(1, O))
    (p2,) = _sc_agg_o(yr2.reshape(2 * N, HW), src4, dst3, z_hw)
    return _tc4_call(p2, dg, yr2)


# speedup vs baseline: 1.0486x; 1.0466x over previous
import jax
import jax.numpy as jnp
import numpy as _np
from jax import lax
from jax.experimental import pallas as pl
from jax.experimental.pallas import tpu as pltpu
from jax.experimental.pallas import tpu_sc as plsc

N, E, D, H, O = 10000, 320000, 128, 128, 64
HW = H // 2
NC, NS = 2, 16
NW = NC * NS
B = 128
EPAD = 323584
EP = EPAD // NW
NB = EP // B
NPAD = 10240
RPT = NPAD // NS
WDEG = 16
NBLK = 2000
DEPTH = 4
F32 = jnp.float32


def _make_sc_agg(two_sweeps, with_deg):
    mesh = plsc.VectorSubcoreMesh(core_axis_name="c", subcore_axis_name="s")
    out_type = [jax.ShapeDtypeStruct((NC, NPAD, H), F32)]
    scratch = [
        pltpu.VMEM((NB, B), jnp.int32),
        pltpu.VMEM((NB, B), jnp.int32),
    ] + [pltpu.VMEM((B, HW), F32)] * DEPTH + [
        pltpu.VMEM_SHARED((NPAD, HW), F32),
    ] + [pltpu.SemaphoreType.DMA] * DEPTH + [
    ]
    if with_deg:
        out_type.append(jax.ShapeDtypeStruct((NC, NPAD, WDEG), F32))
        scratch += [
            pltpu.VMEM((B, WDEG), F32),
            pltpu.VMEM_SHARED((NPAD, WDEG), F32),
        ]

    def body(*refs):
        y, src4, dst3, zw = refs[:4]
        rest = refs[4:]
        if with_deg:
            (zdeg, ones_h, p_out, d_out, sidx, didx) = rest[:6]
            rbufs = rest[6:6 + DEPTH]
            acc = rest[6 + DEPTH]
            sems = rest[7 + DEPTH:7 + 2 * DEPTH]
            ones_v, dacc = rest[7 + 2 * DEPTH:]
        else:
            p_out, sidx, didx = rest[:3]
            rbufs = rest[3:3 + DEPTH]
            acc = rest[3 + DEPTH]
            sems = rest[4 + DEPTH:4 + 2 * DEPTH]
        bufs = tuple(zip(rbufs, sems))
        c = lax.axis_index("c")
        s = lax.axis_index("s")
        wid = s * NC + c
        rows = pl.ds(s * RPT, RPT)

        pltpu.sync_copy(zw, acc.at[rows])
        if with_deg:
            pltpu.sync_copy(zdeg, dacc.at[rows])
            pltpu.sync_copy(ones_h, ones_v)
        pltpu.sync_copy(src4.at[wid], sidx)
        pltpu.sync_copy(dst3.at[wid], didx)
        plsc.subcore_barrier()

        def sweep(count_deg):
            def start(i, buf, sem):
                pltpu.async_copy(y.at[sidx.at[i]], buf, sem)

            def finish(i, buf, sem):
                pltpu.make_async_copy(y.at[sidx.at[i]], buf, sem).wait()
                pltpu.sync_copy(buf, acc.at[didx.at[i]], add=True)
                if count_deg:
                    pltpu.sync_copy(ones_v, dacc.at[didx.at[i]], add=True)

            depth = len(bufs)
            nmain = (NB - (depth - 1)) // depth
            for j in range(depth - 1):
                start(j, *bufs[j])

            def ring(k, carry):
                base = depth * k
                for j in range(depth):
                    i = base + j
                    start(i + depth - 1, *bufs[(j + depth - 1) % depth])
                    finish(i, *bufs[j])
                return carry

            lax.fori_loop(0, nmain, ring, 0)
            for i in range(nmain * depth, NB):
                nxt = i + depth - 1
                if nxt < NB:
                    start(nxt, *bufs[nxt % depth])
                finish(i, *bufs[i % depth])

        sweep(with_deg)
        plsc.subcore_barrier()
        pltpu.sync_copy(acc.at[rows], p_out.at[c, rows, pl.ds(0, HW)])
        if with_deg:
            pltpu.sync_copy(dacc.at[rows], d_out.at[c, rows])
        if two_sweeps:
            pltpu.sync_copy(zw, acc.at[rows])

            def bump(i, carry):
                for k in range(B // 16):
                    sl = (i, pl.ds(16 * k, 16))
                    sidx[sl] = sidx[sl] + 1
                return carry

            lax.fori_loop(0, NB, bump, 0)
            plsc.subcore_barrier()
            sweep(False)
            plsc.subcore_barrier()
            pltpu.sync_copy(acc.at[rows], p_out.at[c, rows, pl.ds(HW, HW)])

    return pl.kernel(body, out_type=out_type, mesh=mesh, scratch_types=scratch,
                     compiler_params=pltpu.CompilerParams(
                         use_tc_tiling_on_sc=False))


_sc_agg_deg = _make_sc_agg(True, True)
_sc_agg_h = _make_sc_agg(True, False)
_sc_agg_o = _make_sc_agg(False, False)


def _row_spec(w):
    return pl.BlockSpec((NBLK, w), lambda i: (i, 0))


def _full_spec(shape):
    nd = len(shape)
    return pl.BlockSpec(shape, lambda i, _nd=nd: (0,) * _nd)


def _part_spec(w):
    return pl.BlockSpec((NC, NBLK, w), lambda i: (0, i, 0))


def _inv_deg(dg_ref):
    deg = dg_ref[0, :, 0:1] + dg_ref[1, :, 0:1]
    return 1.0 / jnp.maximum(deg, 1.0)


def _mm(a, w):
    return lax.dot_general(a, w, (((1,), (1,)), ((), ())),
                           preferred_element_type=F32)


def _tc1(x_r, wl_r, wr_r, bl_r, y_r, r_r):
    xb = x_r[...]
    y_r[...] = _mm(xb, wl_r[...])
    r_r[...] = _mm(xb, wr_r[...]) + bl_r[...]


_tc1_call = pl.pallas_call(
    _tc1,
    grid=(N // NBLK,),
    in_specs=[_row_spec(D), _full_spec((H, D)), _full_spec((H, D)),
              _full_spec((1, H))],
    out_specs=[_row_spec(H), _row_spec(H)],
    out_shape=[jax.ShapeDtypeStruct((N, H), F32)] * 2,
)


def _tc2(p_r, dg_r, r0_r, s_r, b_r, wl_r, wr_r, bl_r, y_r, r_r):
    sage = (p_r[0] + p_r[1]) * _inv_deg(dg_r) + r0_r[...]
    h = jnp.maximum(sage * s_r[...] + b_r[...], 0.0)
    y_r[...] = _mm(h, wl_r[...])
    r_r[...] = _mm(h, wr_r[...]) + bl_r[...]


_tc2_call = pl.pallas_call(
    _tc2,
    grid=(N // NBLK,),
    in_specs=[_part_spec(H), _part_spec(WDEG), _row_spec(H),
              _full_spec((1, H)), _full_spec((1, H)),
              _full_spec((H, H)), _full_spec((H, H)), _full_spec((1, H))],
    out_specs=[_row_spec(H), _row_spec(H)],
    out_shape=[jax.ShapeDtypeStruct((N, H), F32)] * 2,
)


def _tc3(q_r, dg_r, r1_r, s_r, b_r, af_r, wh_r, bh_r,
         wl2_r, wr2_r, bl2_r, yr_r):
    sage = (q_r[0] + q_r[1]) * _inv_deg(dg_r) + r1_r[...]
    h1 = jnp.maximum(sage * s_r[...] + b_r[...], 0.0)
    af = _mm(af_r[...], wh_r[...]) + bh_r[...]
    cat = jnp.concatenate([h1, af], axis=-1)
    yr_r[...] = jnp.concatenate(
        [_mm(cat, wl2_r[...]), _mm(cat, wr2_r[...]) + bl2_r[...]], axis=-1)


_tc3_call = pl.pallas_call(
    _tc3,
    grid=(N // NBLK,),
    in_specs=[_part_spec(H), _part_spec(WDEG), _row_spec(H),
              _full_spec((1, H)), _full_spec((1, H)),
              _row_spec(D), _full_spec((H, D)), _full_spec((1, H)),
              _full_spec((O, 2 * H)), _full_spec((O, 2 * H)),
              _full_spec((1, O))],
    out_specs=_row_spec(H),
    out_shape=jax.ShapeDtypeStruct((N, H), F32),
)


def _tc4(p2_r, dg_r, yr_r, o_r):
    z = ((p2_r[0, :, :O] + p2_r[1, :, :O]) * _inv_deg(dg_r)
         + yr_r[:, O:])
    m = jnp.max(z, axis=-1, keepdims=True)
    ez = jnp.exp(z - m)
    o_r[...] = z - m - jnp.log(jnp.sum(ez, axis=-1, keepdims=True))


_tc4_call = pl.pallas_call(
    _tc4,
    grid=(N // NBLK,),
    in_specs=[_part_spec(H), _part_spec(WDEG), _row_spec(H)],
    out_specs=_row_spec(O),
    out_shape=jax.ShapeDtypeStruct((N, O), F32),
)


def kernel(x, edge_index, Wl0, bl0, Wr0, Wl1, bl1, Wr1, Wl2, bl2, Wr2,
           g0, b0, g1, b1, Wh, bh, add_feat):
    if EPAD != E:
        pad_src = jnp.asarray(_np.arange(EPAD - E, dtype=_np.int32) % N)
        pad_dst = jnp.asarray(
            N + _np.arange(EPAD - E, dtype=_np.int32) % (NPAD - N))
        src = jnp.concatenate([edge_index[0], pad_src])
        dst = jnp.concatenate([edge_index[1], pad_dst])
    else:
        src, dst = edge_index[0], edge_index[1]
    src4 = (src + src).reshape(NW, NB, B)
    dst3 = dst.reshape(NW, NB, B)
    z_hw = jnp.zeros((RPT, HW), F32)
    z_deg = jnp.zeros((RPT, WDEG), F32)
    ones_h = jnp.ones((B, WDEG), F32)
    bn_s0 = (g0 / jnp.sqrt(1.0 + 1e-5)).reshape(1, H)
    bn_s1 = (g1 / jnp.sqrt(1.0 + 1e-5)).reshape(1, H)

    y0, r0 = _tc1_call(x, Wl0, Wr0, bl0.reshape(1, H))
    p0, dg = _sc_agg_deg(y0.reshape(2 * N, HW), src4, dst3, z_hw,
                         z_deg, ones_h)
    y1, r1 = _tc2_call(p0, dg, r0, bn_s0, b0.reshape(1, H),
                       Wl1, Wr1, bl1.reshape(1, H))
    (p1,) = _sc_agg_h(y1.reshape(2 * N, HW), src4, dst3, z_hw)
    yr2 = _tc3_call(p1, dg, r1, bn_s1, b1.reshape(1, H),
                    add_feat, Wh, bh.reshape(1, H),
                    Wl2, Wr2, bl2.reshape(1, O))
    (p2,) = _sc_agg_o(yr2.reshape(2 * N, HW), src4, dst3, z_hw)
    return _tc4_call(p2, dg, yr2)
